# baseline (speedup 1.0000x reference)
import functools

import jax
import jax.numpy as jnp
from jax import lax
from jax.experimental import pallas as pl
from jax.experimental.pallas import tpu as pltpu
from jax.experimental.pallas import tpu_sc as plsc

N = 320000
D = 128
NB = 10000

NC = 2
NS = 16
NW = NC * NS
C = N // NW
NBP = 10240
BPW = NBP // NW
L = 16

RCH = 400
NCHK = C // RCH
GPC = RCH // L

NEG = float("-inf")

_MESH = plsc.VectorSubcoreMesh(core_axis_name="c", subcore_axis_name="s")
_SC_PARAMS = pltpu.CompilerParams(
    needs_layout_passes=False, use_tc_tiling_on_sc=False)


def _take(v, idx):
    return jnp.take_along_axis(v, idx, axis=0, mode="promise_in_bounds")


@functools.partial(
    pl.kernel,
    mesh=_MESH,
    compiler_params=_SC_PARAMS,
    out_type=(
        jax.ShapeDtypeStruct((N,), jnp.float32),
        jax.ShapeDtypeStruct((NW, NBP), jnp.float32),
    ),
    scratch_types=[
        pltpu.VMEM((2 * RCH, D), jnp.float32),
        pltpu.VMEM((2, RCH), jnp.int32),
        pltpu.VMEM((2, RCH), jnp.float32),
        pltpu.VMEM((NBP,), jnp.float32),
        pltpu.VMEM((D,), jnp.float32),
        pltpu.VMEM((L,), jnp.float32),
        pltpu.SemaphoreType.DMA((2,)),
        pltpu.SemaphoreType.DMA((2,)),
        pltpu.SemaphoreType.DMA((2,)),
    ],
)
def _sc_fused(z_hbm, seg_hbm, w_hbm, b_hbm, out_s_hbm, out_m_hbm,
              zb, segb, sb, m_v, w_v, b_v, zsem, gsem, osem):
    wid = lax.axis_index("s") * NC + lax.axis_index("c")
    base = pl.multiple_of(wid * C, 8)

    pltpu.sync_copy(w_hbm, w_v)
    pltpu.sync_copy(b_hbm, b_v)

    def z_dma(chunk, slot):
        return pltpu.make_async_copy(
            z_hbm.at[pl.ds(base + chunk * RCH, RCH), :],
            zb.at[pl.ds(slot * RCH, RCH), :], zsem.at[slot])

    def seg_dma(chunk, slot):
        return pltpu.make_async_copy(
            seg_hbm.at[pl.ds(base + chunk * RCH, RCH)],
            segb.at[slot], gsem.at[slot])

    def out_dma(chunk, slot):
        return pltpu.make_async_copy(
            sb.at[slot],
            out_s_hbm.at[pl.ds(base + chunk * RCH, RCH)], osem.at[slot])

    z_dma(0, 0).start()
    seg_dma(0, 0).start()
    z_dma(1, 1).start()
    seg_dma(1, 1).start()

    neg = jnp.full((L,), NEG, jnp.float32)

    def init_body(i, carry):
        m_v[pl.ds(pl.multiple_of(i * L, L), L)] = neg
        return carry

    lax.fori_loop(0, NBP // L, init_body, 0, unroll=8)

    wv = [w_v[pl.ds(16 * j, L)] for j in range(D // L)]
    bvec = b_v[...]
    lane = lax.iota(jnp.int32, L)
    last_lane = lane == (L - 1)
    up1 = jnp.minimum(lane + 1, L - 1)

    def chunk_body(chunk, carry):
        slot = lax.rem(chunk, 2)

        for s in (0, 1):
            @pl.when(slot == s)
            def _():
                z_dma(chunk, s).wait()
                seg_dma(chunk, s).wait()

            @pl.when((slot == s) & (chunk >= 2))
            def _():
                out_dma(chunk - 2, s).wait()

        def group_body(gin, carry2):
            rowbase = slot * RCH + gin * L
            v = neg
            for l in range(L):
                row = rowbase + l
                prods = [zb[row, pl.ds(16 * j, L)] * wv[j]
                         for j in range(D // L)]
                while len(prods) > 1:
                    prods = [a + b for a, b in zip(prods[::2], prods[1::2])]
                v = jnp.where(lane == l, jnp.sum(prods[0]), v)
            v = v + bvec
            sb[slot, pl.ds(pl.multiple_of(gin * L, L), L)] = v

            gid = segb[slot, pl.ds(pl.multiple_of(gin * L, L), L)]
            for s in (1, 2, 4, 8):
                idx = jnp.maximum(lane - s, 0)
                vs = _take(v, idx)
                gs = _take(gid, idx)
                v = jnp.where((gs == gid) & (lane >= s),
                              jnp.maximum(v, vs), v)
            g_next = _take(gid, up1)
            is_last = (g_next != gid) | last_lane
            cur = plsc.load_gather(m_v, [gid], mask=is_last)
            plsc.store_scatter(m_v, [gid], jnp.maximum(cur, v), mask=is_last)
            return carry2

        lax.fori_loop(0, GPC, group_body, 0, unroll=2)

        for s in (0, 1):
            @pl.when(slot == s)
            def _():
                out_dma(chunk, s).start()

            @pl.when((slot == s) & (chunk + 2 < NCHK))
            def _():
                z_dma(chunk + 2, s).start()
                seg_dma(chunk + 2, s).start()
        return carry

    lax.fori_loop(0, NCHK, chunk_body, 0)

    out_dma(NCHK - 2, (NCHK - 2) % 2).wait()
    out_dma(NCHK - 1, (NCHK - 1) % 2).wait()
    pltpu.sync_copy(m_v, out_m_hbm.at[wid])


BPW_LAST = NB - (NW - 1) * BPW


@functools.partial(
    pl.kernel,
    mesh=_MESH,
    compiler_params=_SC_PARAMS,
    out_type=jax.ShapeDtypeStruct((NB,), jnp.float32),
    scratch_types=[
        pltpu.VMEM((NW, BPW), jnp.float32),
        pltpu.VMEM((BPW,), jnp.float32),
    ],
)
def _segmax_merge(parts_hbm, out_hbm, blk_v, acc_v):
    wid = lax.axis_index("s") * NC + lax.axis_index("c")
    lo = pl.multiple_of(wid * BPW, 8)
    pltpu.sync_copy(parts_hbm.at[:, pl.ds(lo, BPW)], blk_v)

    def body(j, carry):
        off = pl.multiple_of(j * L, L)
        acc = jnp.full((L,), NEG, jnp.float32)
        for r in range(NW):
            acc = jnp.maximum(acc, blk_v[r, pl.ds(off, L)])
        acc_v[pl.ds(off, L)] = acc
        return carry

    @pl.when(wid < NW - 1)
    def _():
        lax.fori_loop(0, BPW // L, body, 0)
        pltpu.sync_copy(acc_v, out_hbm.at[pl.ds(lo, BPW)])

    @pl.when(wid == NW - 1)
    def _():
        lax.fori_loop(0, BPW_LAST // L, body, 0)
        pltpu.sync_copy(acc_v.at[pl.ds(0, BPW_LAST)],
                        out_hbm.at[pl.ds(lo, BPW_LAST)])


TBLK = 12800
TROWS = 102400
TOFF = N - TROWS


def _tc_matvec_body(z_ref, w_ref, b_ref, out_ref):
    x = z_ref[...]
    w = w_ref[...]
    s = jax.lax.dot_general(
        x, w, (((1,), (0,)), ((), ())),
        preferred_element_type=jnp.float32)
    out_ref[...] = s + b_ref[0, 0]


def _tc_scores_tail(z, W, b):
    wcol = W.reshape(D, 1)
    b2 = b.reshape(1, 1)
    out = pl.pallas_call(
        _tc_matvec_body,
        grid=(TROWS // TBLK,),
        in_specs=[
            pl.BlockSpec((TBLK, D), lambda i: (TOFF // TBLK + i, 0)),
            pl.BlockSpec((D, 1), lambda i: (0, 0)),
            pl.BlockSpec((1, 1), lambda i: (0, 0)),
        ],
        out_specs=pl.BlockSpec((TBLK, 1), lambda i: (i, 0)),
        out_shape=jax.ShapeDtypeStruct((TROWS, 1), jnp.float32),
    )(z, wcol, b2)
    return out.reshape(TROWS)


def kernel(z_ins, bag_idx, W, b):
    seg = bag_idx.astype(jnp.int32)
    w1 = W.reshape(D)
    b16 = jnp.broadcast_to(b, (L,))
    scores, parts = _sc_fused(z_ins, seg, w1, b16)
    scores_tail = _tc_scores_tail(z_ins, W, b)
    M = _segmax_merge(parts)[:, None]
    scores = jnp.concatenate([scores[:TOFF], scores_tail], axis=0)
    return (M, None, scores)


# speedup vs baseline: 1.4720x; 1.4720x over previous
import functools

import jax
import jax.numpy as jnp
from jax import lax
from jax.experimental import pallas as pl
from jax.experimental.pallas import tpu as pltpu
from jax.experimental.pallas import tpu_sc as plsc

N = 320000
D = 128
NB = 10000

NC = 2
NS = 16
NW = NC * NS
C = N // NW
NBP = 10240
BPW = NBP // NW
L = 16

RCH = 400
NCHK = C // RCH
GPC = RCH // L

NEG = float("-inf")

_MESH = plsc.VectorSubcoreMesh(core_axis_name="c", subcore_axis_name="s")
_SC_PARAMS = pltpu.CompilerParams(
    needs_layout_passes=False, use_tc_tiling_on_sc=False)


def _take(v, idx):
    return jnp.take_along_axis(v, idx, axis=0, mode="promise_in_bounds")


@functools.partial(
    pl.kernel,
    mesh=_MESH,
    compiler_params=_SC_PARAMS,
    out_type=(
        jax.ShapeDtypeStruct((N,), jnp.float32),
        jax.ShapeDtypeStruct((NW, NBP), jnp.float32),
    ),
    scratch_types=[
        pltpu.VMEM((2 * RCH, D), jnp.float32),
        pltpu.VMEM((2, RCH), jnp.int32),
        pltpu.VMEM((2, RCH), jnp.float32),
        pltpu.VMEM((NBP,), jnp.float32),
        pltpu.VMEM((D,), jnp.float32),
        pltpu.VMEM((L,), jnp.float32),
        pltpu.SemaphoreType.DMA((2,)),
        pltpu.SemaphoreType.DMA((2,)),
        pltpu.SemaphoreType.DMA((2,)),
    ],
)
def _sc_fused(z_hbm, seg_hbm, w_hbm, b_hbm, out_s_hbm, out_m_hbm,
              zb, segb, sb, m_v, w_v, b_v, zsem, gsem, osem):
    wid = lax.axis_index("s") * NC + lax.axis_index("c")
    base = pl.multiple_of(wid * C, 8)

    pltpu.sync_copy(w_hbm, w_v)
    pltpu.sync_copy(b_hbm, b_v)

    def z_dma(chunk, slot):
        return pltpu.make_async_copy(
            z_hbm.at[pl.ds(base + chunk * RCH, RCH), :],
            zb.at[pl.ds(slot * RCH, RCH), :], zsem.at[slot])

    def seg_dma(chunk, slot):
        return pltpu.make_async_copy(
            seg_hbm.at[pl.ds(base + chunk * RCH, RCH)],
            segb.at[slot], gsem.at[slot])

    def out_dma(chunk, slot):
        return pltpu.make_async_copy(
            sb.at[slot],
            out_s_hbm.at[pl.ds(base + chunk * RCH, RCH)], osem.at[slot])

    z_dma(0, 0).start()
    seg_dma(0, 0).start()
    z_dma(1, 1).start()
    seg_dma(1, 1).start()

    neg = jnp.full((L,), NEG, jnp.float32)

    def init_body(i, carry):
        m_v[pl.ds(pl.multiple_of(i * L, L), L)] = neg
        return carry

    lax.fori_loop(0, NBP // L, init_body, 0, unroll=8)

    wv = [w_v[pl.ds(16 * j, L)] for j in range(D // L)]
    bvec = b_v[...]
    lane = lax.iota(jnp.int32, L)
    last_lane = lane == (L - 1)
    up1 = jnp.minimum(lane + 1, L - 1)

    def chunk_body(chunk, carry):
        slot = lax.rem(chunk, 2)

        for s in (0, 1):
            @pl.when(slot == s)
            def _():
                z_dma(chunk, s).wait()
                seg_dma(chunk, s).wait()

            @pl.when((slot == s) & (chunk >= 2))
            def _():
                out_dma(chunk - 2, s).wait()

        def group_body(gin, carry2):
            rowbase = slot * RCH + gin * L
            v = neg
            for l in range(L):
                row = rowbase + l
                prods = [zb[row, pl.ds(16 * j, L)] * wv[j]
                         for j in range(D // L)]
                while len(prods) > 1:
                    prods = [a + b for a, b in zip(prods[::2], prods[1::2])]
                v = jnp.where(lane == l, jnp.sum(prods[0]), v)
            v = v + bvec
            sb[slot, pl.ds(pl.multiple_of(gin * L, L), L)] = v

            gid = segb[slot, pl.ds(pl.multiple_of(gin * L, L), L)]
            for s in (1, 2, 4, 8):
                idx = jnp.maximum(lane - s, 0)
                vs = _take(v, idx)
                gs = _take(gid, idx)
                v = jnp.where((gs == gid) & (lane >= s),
                              jnp.maximum(v, vs), v)
            g_next = _take(gid, up1)
            is_last = (g_next != gid) | last_lane
            cur = plsc.load_gather(m_v, [gid], mask=is_last)
            plsc.store_scatter(m_v, [gid], jnp.maximum(cur, v), mask=is_last)
            return carry2

        lax.fori_loop(0, GPC, group_body, 0, unroll=2)

        for s in (0, 1):
            @pl.when(slot == s)
            def _():
                out_dma(chunk, s).start()

            @pl.when((slot == s) & (chunk + 2 < NCHK))
            def _():
                z_dma(chunk + 2, s).start()
                seg_dma(chunk + 2, s).start()
        return carry

    lax.fori_loop(0, NCHK, chunk_body, 0)

    out_dma(NCHK - 2, (NCHK - 2) % 2).wait()
    out_dma(NCHK - 1, (NCHK - 1) % 2).wait()
    pltpu.sync_copy(m_v, out_m_hbm.at[wid])


BPW_LAST = NB - (NW - 1) * BPW


@functools.partial(
    pl.kernel,
    mesh=_MESH,
    compiler_params=_SC_PARAMS,
    out_type=jax.ShapeDtypeStruct((NB,), jnp.float32),
    scratch_types=[
        pltpu.VMEM((NW, BPW), jnp.float32),
        pltpu.VMEM((BPW,), jnp.float32),
    ],
)
def _segmax_merge(parts_hbm, out_hbm, blk_v, acc_v):
    wid = lax.axis_index("s") * NC + lax.axis_index("c")
    lo = pl.multiple_of(wid * BPW, 8)
    pltpu.sync_copy(parts_hbm.at[:, pl.ds(lo, BPW)], blk_v)

    def body(j, carry):
        off = pl.multiple_of(j * L, L)
        acc = jnp.full((L,), NEG, jnp.float32)
        for r in range(NW):
            acc = jnp.maximum(acc, blk_v[r, pl.ds(off, L)])
        acc_v[pl.ds(off, L)] = acc
        return carry

    @pl.when(wid < NW - 1)
    def _():
        lax.fori_loop(0, BPW // L, body, 0)
        pltpu.sync_copy(acc_v, out_hbm.at[pl.ds(lo, BPW)])

    @pl.when(wid == NW - 1)
    def _():
        lax.fori_loop(0, BPW_LAST // L, body, 0)
        pltpu.sync_copy(acc_v.at[pl.ds(0, BPW_LAST)],
                        out_hbm.at[pl.ds(lo, BPW_LAST)])


def kernel(z_ins, bag_idx, W, b):
    seg = bag_idx.astype(jnp.int32)
    w1 = W.reshape(D)
    b16 = jnp.broadcast_to(b, (L,))
    scores, parts = _sc_fused(z_ins, seg, w1, b16)
    M = _segmax_merge(parts)[:, None]
    return (M, None, scores)


# speedup vs baseline: 1.4779x; 1.0040x over previous
import functools

import jax
import jax.numpy as jnp
from jax import lax
from jax.experimental import pallas as pl
from jax.experimental.pallas import tpu as pltpu
from jax.experimental.pallas import tpu_sc as plsc

N = 320000
D = 128
NB = 10000

NC = 2
NS = 16
NW = NC * NS
C = N // NW
NBP = 10240
BPW = NBP // NW
L = 16

RCH = 400
NCHK = C // RCH
GPC = RCH // L

NEG = float("-inf")

_MESH = plsc.VectorSubcoreMesh(core_axis_name="c", subcore_axis_name="s")
_SC_PARAMS = pltpu.CompilerParams(
    needs_layout_passes=False, use_tc_tiling_on_sc=False)


def _take(v, idx):
    return jnp.take_along_axis(v, idx, axis=0, mode="promise_in_bounds")


@functools.partial(
    pl.kernel,
    mesh=_MESH,
    compiler_params=_SC_PARAMS,
    out_type=(
        jax.ShapeDtypeStruct((N,), jnp.float32),
        jax.ShapeDtypeStruct((NW, NBP), jnp.float32),
    ),
    scratch_types=[
        pltpu.VMEM((2 * RCH, D), jnp.float32),
        pltpu.VMEM((2, RCH), jnp.int32),
        pltpu.VMEM((2, RCH), jnp.float32),
        pltpu.VMEM((NBP,), jnp.float32),
        pltpu.VMEM((D,), jnp.float32),
        pltpu.VMEM((L,), jnp.float32),
        pltpu.SemaphoreType.DMA((2,)),
        pltpu.SemaphoreType.DMA((2,)),
        pltpu.SemaphoreType.DMA((2,)),
    ],
)
def _sc_fused(z_hbm, seg_hbm, w_hbm, b_hbm, out_s_hbm, out_m_hbm,
              zb, segb, sb, m_v, w_v, b_v, zsem, gsem, osem):
    wid = lax.axis_index("s") * NC + lax.axis_index("c")
    base = pl.multiple_of(wid * C, 8)

    pltpu.sync_copy(w_hbm.at[0], w_v)
    pltpu.sync_copy(b_hbm, b_v)

    def z_dma(chunk, slot):
        return pltpu.make_async_copy(
            z_hbm.at[pl.ds(base + chunk * RCH, RCH), :],
            zb.at[pl.ds(slot * RCH, RCH), :], zsem.at[slot])

    def seg_dma(chunk, slot):
        return pltpu.make_async_copy(
            seg_hbm.at[pl.ds(base + chunk * RCH, RCH)],
            segb.at[slot], gsem.at[slot])

    def out_dma(chunk, slot):
        return pltpu.make_async_copy(
            sb.at[slot],
            out_s_hbm.at[pl.ds(base + chunk * RCH, RCH)], osem.at[slot])

    z_dma(0, 0).start()
    seg_dma(0, 0).start()
    z_dma(1, 1).start()
    seg_dma(1, 1).start()

    neg = jnp.full((L,), NEG, jnp.float32)

    def init_body(i, carry):
        m_v[pl.ds(pl.multiple_of(i * L, L), L)] = neg
        return carry

    lax.fori_loop(0, NBP // L, init_body, 0, unroll=8)

    wv = [w_v[pl.ds(16 * j, L)] for j in range(D // L)]
    bvec = b_v[...]
    lane = lax.iota(jnp.int32, L)
    last_lane = lane == (L - 1)
    up1 = jnp.minimum(lane + 1, L - 1)

    def chunk_body(chunk, carry):
        slot = lax.rem(chunk, 2)

        for s in (0, 1):
            @pl.when(slot == s)
            def _():
                z_dma(chunk, s).wait()
                seg_dma(chunk, s).wait()

            @pl.when((slot == s) & (chunk >= 2))
            def _():
                out_dma(chunk - 2, s).wait()

        def group_body(gin, carry2):
            rowbase = slot * RCH + gin * L
            v = neg
            for l in range(L):
                row = rowbase + l
                prods = [zb[row, pl.ds(16 * j, L)] * wv[j]
                         for j in range(D // L)]
                while len(prods) > 1:
                    prods = [a + b for a, b in zip(prods[::2], prods[1::2])]
                v = jnp.where(lane == l, jnp.sum(prods[0]), v)
            v = v + bvec
            sb[slot, pl.ds(pl.multiple_of(gin * L, L), L)] = v

            gid = segb[slot, pl.ds(pl.multiple_of(gin * L, L), L)]
            for s in (1, 2, 4, 8):
                idx = jnp.maximum(lane - s, 0)
                vs = _take(v, idx)
                gs = _take(gid, idx)
                v = jnp.where((gs == gid) & (lane >= s),
                              jnp.maximum(v, vs), v)
            g_next = _take(gid, up1)
            is_last = (g_next != gid) | last_lane
            cur = plsc.load_gather(m_v, [gid], mask=is_last)
            plsc.store_scatter(m_v, [gid], jnp.maximum(cur, v), mask=is_last)
            return carry2

        lax.fori_loop(0, GPC, group_body, 0, unroll=2)

        for s in (0, 1):
            @pl.when(slot == s)
            def _():
                out_dma(chunk, s).start()

            @pl.when((slot == s) & (chunk + 2 < NCHK))
            def _():
                z_dma(chunk + 2, s).start()
                seg_dma(chunk + 2, s).start()
        return carry

    lax.fori_loop(0, NCHK, chunk_body, 0)

    out_dma(NCHK - 2, (NCHK - 2) % 2).wait()
    out_dma(NCHK - 1, (NCHK - 1) % 2).wait()
    pltpu.sync_copy(m_v, out_m_hbm.at[wid])


BPW_LAST = NB - (NW - 1) * BPW


@functools.partial(
    pl.kernel,
    mesh=_MESH,
    compiler_params=_SC_PARAMS,
    out_type=jax.ShapeDtypeStruct((NB,), jnp.float32),
    scratch_types=[
        pltpu.VMEM((NW, BPW), jnp.float32),
        pltpu.VMEM((BPW,), jnp.float32),
    ],
)
def _segmax_merge(parts_hbm, out_hbm, blk_v, acc_v):
    wid = lax.axis_index("s") * NC + lax.axis_index("c")
    lo = pl.multiple_of(wid * BPW, 8)
    pltpu.sync_copy(parts_hbm.at[:, pl.ds(lo, BPW)], blk_v)

    def body(j, carry):
        off = pl.multiple_of(j * L, L)
        acc = jnp.full((L,), NEG, jnp.float32)
        for r in range(NW):
            acc = jnp.maximum(acc, blk_v[r, pl.ds(off, L)])
        acc_v[pl.ds(off, L)] = acc
        return carry

    @pl.when(wid < NW - 1)
    def _():
        lax.fori_loop(0, BPW // L, body, 0)
        pltpu.sync_copy(acc_v, out_hbm.at[pl.ds(lo, BPW)])

    @pl.when(wid == NW - 1)
    def _():
        lax.fori_loop(0, BPW_LAST // L, body, 0)
        pltpu.sync_copy(acc_v.at[pl.ds(0, BPW_LAST)],
                        out_hbm.at[pl.ds(lo, BPW_LAST)])


def kernel(z_ins, bag_idx, W, b):
    seg = bag_idx.astype(jnp.int32)
    b16 = jnp.broadcast_to(b, (L,))
    scores, parts = _sc_fused(z_ins, seg, W, b16)
    M = _segmax_merge(parts)[:, None]
    return (M, None, scores)
